# baseline (speedup 1.0000x reference)
import jax
import jax.numpy as jnp
from jax.experimental import pallas as pl
from jax.experimental.pallas import tpu as pltpu

K = 16
D = 128


def _select_body(conf_ref, dens_ref, out_ref, sem):
    i = conf_ref[0]
    copy = pltpu.make_async_copy(dens_ref.at[i], out_ref, sem)
    copy.start()
    copy.wait()


def kernel(density, conformation):
    conf = jnp.atleast_1d(jnp.asarray(conformation, jnp.int32))
    grid_spec = pltpu.PrefetchScalarGridSpec(
        num_scalar_prefetch=1,
        grid=(1,),
        in_specs=[pl.BlockSpec(memory_space=pl.ANY)],
        out_specs=pl.BlockSpec(memory_space=pl.ANY),
        scratch_shapes=[pltpu.SemaphoreType.DMA],
    )
    return pl.pallas_call(
        _select_body,
        grid_spec=grid_spec,
        out_shape=jax.ShapeDtypeStruct((D, D, D), density.dtype),
    )(conf, density)
